# Initial kernel scaffold; baseline (speedup 1.0000x reference)
import jax
import jax.numpy as jnp
from jax.experimental import pallas as pl


def kernel(x, t, emb_table, W, b):
    raise NotImplementedError("write your pallas kernel here")


# speedup vs baseline: 2.4413x; 2.4413x over previous
import functools

import jax
import jax.numpy as jnp
from jax import lax
from jax.experimental import pallas as pl
from jax.experimental.pallas import tpu as pltpu
from jax.experimental.pallas import tpu_sc as plsc

_H = 64
_NC = 2
_NS = 16
_NW = _NC * _NS
_CHUNK = 512


def _proj_body(emb_ref, w_ref, b_ref, out_ref):
    out_ref[...] = lax.dot_general(
        emb_ref[...], w_ref[...], (((1,), (1,)), ((), ())),
        preferred_element_type=jnp.float32) + b_ref[...]


def _make_sc_call(n_rows: int):
    rows_per_w = n_rows // _NW
    n_chunks = rows_per_w // _CHUNK

    def _sc_body(p_hbm, x_hbm, t_hbm, out_hbm, p_v, t_v, buf):
        wid = lax.axis_index("s") * _NC + lax.axis_index("c")
        base = wid * rows_per_w
        pltpu.sync_copy(p_hbm, p_v)

        def chunk_body(g, carry):
            start = base + g * _CHUNK
            pltpu.sync_copy(t_hbm.at[pl.ds(start, _CHUNK)], t_v)
            pltpu.sync_copy(x_hbm.at[pl.ds(start, _CHUNK)], buf)

            def row_body(i, c2):
                tvec = t_v[pl.ds(i * 16, 16)]
                for k in range(16):
                    ti = tvec[k]
                    for cg in range(_H // 16):
                        plsc.addupdate(buf.at[i * 16 + k, pl.ds(cg * 16, 16)],
                                       p_v[ti, pl.ds(cg * 16, 16)])
                return c2

            lax.fori_loop(0, _CHUNK // 16, row_body, 0)
            pltpu.sync_copy(buf, out_hbm.at[pl.ds(start, _CHUNK)])
            return carry

        lax.fori_loop(0, n_chunks, chunk_body, 0)

    return pl.kernel(
        _sc_body,
        out_type=jax.ShapeDtypeStruct((n_rows, _H), jnp.float32),
        mesh=plsc.VectorSubcoreMesh(core_axis_name="c", subcore_axis_name="s"),
        scratch_types=[
            pltpu.VMEM((_H, _H), jnp.float32),
            pltpu.VMEM((_CHUNK,), jnp.int32),
            pltpu.VMEM((_CHUNK, _H), jnp.float32),
        ],
    )


def kernel(x, t, emb_table, W, b):
    batch, hist, h = x.shape
    n_rows = batch * hist
    p = pl.pallas_call(
        _proj_body,
        out_shape=jax.ShapeDtypeStruct((_H, _H), jnp.float32),
    )(emb_table, W, b.reshape(1, _H))
    out = _make_sc_call(n_rows)(p, x.reshape(n_rows, h), t.reshape(n_rows))
    return out.reshape(x.shape)
